# baseline (speedup 1.0000x reference)
import functools

import jax
import jax.numpy as jnp
from jax import lax
from jax.experimental import pallas as pl
from jax.experimental.pallas import tpu as pltpu
from jax.experimental.pallas import tpu_sc as plsc

VOCAB = 50257
HIDDEN = 768
NC = 2
NS = 16
NW = NC * NS
CH = 32
NBUF = 5
DEPTH = 4

_mesh = plsc.VectorSubcoreMesh(core_axis_name="c", subcore_axis_name="s")


def _make_gather(n_total: int):
  assert n_total % NW == 0
  bpw = n_total // NW
  assert bpw % CH == 0
  nch = bpw // CH

  @functools.partial(
      pl.kernel,
      mesh=_mesh,
      out_type=jax.ShapeDtypeStruct((NW, nch, CH, HIDDEN), jnp.float32),
      scratch_types=[
          pltpu.VMEM((nch, CH), jnp.int32),
          pltpu.VMEM((NBUF, CH, HIDDEN), jnp.float32),
          *([pltpu.SemaphoreType.DMA] * 10),
      ],
  )
  def gather_kernel(table_hbm, ids_hbm, out_hbm, idx_v, rows_v, *sems):
    wid = lax.axis_index("s") * NC + lax.axis_index("c")
    pltpu.sync_copy(ids_hbm.at[wid], idx_v)

    gsems = sems[:NBUF]
    osems = sems[NBUF:]
    cps = [None] * nch
    ocs = [None] * nch
    for k in range(DEPTH):
      cps[k] = pltpu.async_copy(
          table_hbm.at[idx_v.at[k]], rows_v.at[k], gsems[k])
    for g in range(nch):
      b = g % NBUF
      cps[g].wait()
      ocs[g] = pltpu.async_copy(rows_v.at[b], out_hbm.at[wid, g], osems[b])
      nxt = g + DEPTH
      if nxt < nch:
        if nxt - NBUF >= 0:
          ocs[nxt - NBUF].wait()
        cps[nxt] = pltpu.async_copy(
            table_hbm.at[idx_v.at[nxt]], rows_v.at[nxt % NBUF],
            gsems[nxt % NBUF])
    for g in range(nch - NBUF, nch):
      ocs[g].wait()

  return gather_kernel, nch


def kernel(input_ids, word_embeddings):
  b, s = input_ids.shape
  n = b * s
  gather, nch = _make_gather(n)
  ids = input_ids.reshape(NW, nch, CH).astype(jnp.int32)
  out = gather(word_embeddings, ids)
  return out.reshape(b, s, HIDDEN)


# speedup vs baseline: 1.0217x; 1.0217x over previous
import functools

import jax
import jax.numpy as jnp
from jax import lax
from jax.experimental import pallas as pl
from jax.experimental.pallas import tpu as pltpu
from jax.experimental.pallas import tpu_sc as plsc

HIDDEN = 768
NC = 2
NS = 16
NW = NC * NS
CH = 32
NBUF = 4
DEPTH = 3

_mesh = plsc.VectorSubcoreMesh(core_axis_name="c", subcore_axis_name="s")


def _make_gather(b: int, s: int):
  n_total = b * s
  assert n_total % NW == 0
  bpw = n_total // NW
  assert bpw % CH == 0 and s % bpw == 0
  nch = bpw // CH
  per_row = s // bpw

  @functools.partial(
      pl.kernel,
      mesh=_mesh,
      out_type=jax.ShapeDtypeStruct((NW, nch, CH, HIDDEN), jnp.float32),
      scratch_types=[
          pltpu.VMEM((bpw,), jnp.int32),
          pltpu.VMEM((NBUF, CH, HIDDEN), jnp.float32),
          *([pltpu.SemaphoreType.DMA] * (2 * NBUF)),
      ],
  )
  def gather_kernel(table_hbm, ids_hbm, out_hbm, idx_v, rows_v, *sems):
    wid = lax.axis_index("s") * NC + lax.axis_index("c")
    gsems = sems[:NBUF]
    osems = sems[NBUF:]
    pltpu.sync_copy(
        ids_hbm.at[wid // per_row, pl.ds((wid % per_row) * bpw, bpw)], idx_v)

    def g_desc(g, j):
      return pltpu.make_async_copy(
          table_hbm.at[idx_v.at[pl.ds(g * CH, CH)]], rows_v.at[j], gsems[j])

    def w_desc(g, j):
      return pltpu.make_async_copy(rows_v.at[j], out_hbm.at[wid, g], osems[j])

    for j in range(DEPTH):
      g_desc(j, j).start()

    def chunk(g, j, drain, refill):
      g_desc(g, j).wait()
      w_desc(g, j).start()
      jn = (j + DEPTH) % NBUF
      if drain:
        w_desc(g, jn).wait()
      if refill:
        g_desc(g + DEPTH, jn).start()

    for j in range(NBUF):
      chunk(j, j, drain=j >= 1, refill=True)

    def turn(t, carry):
      for j in range(NBUF):
        chunk(t * NBUF + j, j, drain=True, refill=True)
      return carry

    lax.fori_loop(1, nch // NBUF - 1, turn, 0)

    for j in range(NBUF):
      g = nch - NBUF + j
      chunk(g, j, drain=True, refill=g + DEPTH < nch)
    w_desc(nch - 1, (nch - 1) % NBUF).wait()

  return gather_kernel, nch


def kernel(input_ids, word_embeddings):
  b, s = input_ids.shape
  gather, nch = _make_gather(b, s)
  out = gather(word_embeddings, input_ids.astype(jnp.int32))
  return out.reshape(b, s, HIDDEN)
